# baseline (speedup 1.0000x reference)
import functools

import jax
import jax.numpy as jnp
from jax.experimental import pallas as pl

SPARSE_TH = 0.01
TH = 14


def _conv_block(x_lo, x_hi, w_ref, b_ref, out_ref):
    xb = jnp.concatenate([x_lo[0].astype(jnp.bfloat16),
                          x_hi[0, 0].astype(jnp.bfloat16)],
                         axis=0)
    w = w_ref[...]
    w = jnp.where(jnp.abs(w) < SPARSE_TH, jnp.zeros_like(w), w)
    wb = w.astype(jnp.bfloat16)
    acc = jnp.zeros((TH * 222, 192), jnp.float32)
    for kh in range(3):
        for kw in range(3):
            xs = xb[kh:kh + TH, kw:kw + 222, :].reshape(TH * 222, 96)
            acc = acc + jnp.dot(xs, wb[kh, kw],
                                preferred_element_type=jnp.float32)
    out_ref[0] = acc.reshape(TH, 222, 192) + b_ref[0]


@functools.partial(jax.jit, static_argnames=("interpret",))
def kernel(input, weight, bias, interpret=False):
    B, H, W, Cin = input.shape
    Cout = weight.shape[0]
    OH, OW = H - 2, W - 2
    nh = H // TH
    w_t = jnp.transpose(weight, (2, 3, 1, 0))
    b2 = bias.reshape(1, Cout)
    xr = input.reshape(B, nh, TH, W, Cin)
    halo = jnp.concatenate([xr[:, 1:, :2], xr[:, -1:, TH - 2:]], axis=1)

    grid = (B, nh)
    out = pl.pallas_call(
        _conv_block,
        grid=grid,
        in_specs=[
            pl.BlockSpec((1, TH, W, Cin), lambda b, i: (b, i, 0, 0)),
            pl.BlockSpec((1, 1, 2, W, Cin), lambda b, i: (b, i, 0, 0, 0)),
            pl.BlockSpec((3, 3, Cin, Cout), lambda b, i: (0, 0, 0, 0)),
            pl.BlockSpec((1, Cout), lambda b, i: (0, 0)),
        ],
        out_specs=pl.BlockSpec((1, TH, OW, Cout), lambda b, i: (b, i, 0, 0)),
        out_shape=jax.ShapeDtypeStruct((B, OH, OW, Cout), jnp.float32),
        interpret=interpret,
    )(input, halo, w_t, b2)
    return out


# speedup vs baseline: 6.0270x; 6.0270x over previous
import functools

import jax
import jax.numpy as jnp
from jax.experimental import pallas as pl

SPARSE_TH = 0.01
TH = 28
KH = KW = 3


def _wprep(w_ref, o_ref):
    w = w_ref[...]
    o_ref[...] = jnp.where(jnp.abs(w) < SPARSE_TH,
                           jnp.zeros_like(w), w).astype(jnp.bfloat16)


def _conv_block(x_lo, x_hi, w_ref, b_ref, out_ref):
    cin, w = x_lo.shape[2], x_lo.shape[3]
    ow = w - 2
    xb = jnp.concatenate([x_lo[0].astype(jnp.bfloat16),
                          x_hi[0, 0].astype(jnp.bfloat16)],
                         axis=0)
    wb = w_ref[...]
    bias = b_ref[...]

    pieces = []
    for r in range(TH + 2):
        row = xb[r]
        for kw in range(KW):
            pieces.append(row[:, kw:kw + ow])
    x_all = jnp.concatenate(pieces, axis=0)

    kdim = KH * KW * cin
    step = KW * cin
    for r in range(TH):
        rhs = x_all[r * step:r * step + kdim]
        acc = jnp.dot(wb, rhs, preferred_element_type=jnp.float32)
        out_ref[0, r] = acc + bias


@functools.partial(jax.jit, static_argnames=("interpret",))
def kernel(input, weight, bias, interpret=False):
    B, H, W, Cin = input.shape
    Cout = weight.shape[0]
    OH, OW = H - 2, W - 2
    nh = H // TH

    xt = jnp.transpose(input, (0, 1, 3, 2))
    wcat = jnp.transpose(weight, (0, 2, 3, 1)).reshape(Cout, KH * KW * Cin)
    wb = pl.pallas_call(
        _wprep,
        out_shape=jax.ShapeDtypeStruct((Cout, KH * KW * Cin), jnp.bfloat16),
        interpret=interpret,
    )(wcat)
    b2 = bias.reshape(Cout, 1)
    xr = xt.reshape(B, nh, TH, Cin, W)
    halo = jnp.concatenate([xr[:, 1:, :2], xr[:, -1:, TH - 2:]], axis=1)

    grid = (B, nh)
    outt = pl.pallas_call(
        _conv_block,
        grid=grid,
        in_specs=[
            pl.BlockSpec((1, TH, Cin, W), lambda b, i: (b, i, 0, 0)),
            pl.BlockSpec((1, 1, 2, Cin, W), lambda b, i: (b, i, 0, 0, 0)),
            pl.BlockSpec((Cout, KH * KW * Cin), lambda b, i: (0, 0)),
            pl.BlockSpec((Cout, 1), lambda b, i: (0, 0)),
        ],
        out_specs=pl.BlockSpec((1, TH, Cout, OW), lambda b, i: (b, i, 0, 0)),
        out_shape=jax.ShapeDtypeStruct((B, OH, Cout, OW), jnp.float32),
        interpret=interpret,
    )(xt, halo, wb, b2)
    return jnp.transpose(outt, (0, 1, 3, 2))
